# Initial kernel scaffold; baseline (speedup 1.0000x reference)
import jax
import jax.numpy as jnp
from jax.experimental import pallas as pl


def kernel(table, ids):
    raise NotImplementedError("write your pallas kernel here")


# speedup vs baseline: 1.5125x; 1.5125x over previous
import functools

import jax
import jax.numpy as jnp
from jax import lax
from jax.experimental import pallas as pl
from jax.experimental.pallas import tpu as pltpu
from jax.experimental.pallas import tpu_sc as plsc

EMBED_DIM = 64
_NIDS = 8

_mesh = plsc.VectorSubcoreMesh(core_axis_name="c", subcore_axis_name="s")


@functools.partial(
    pl.kernel,
    mesh=_mesh,
    out_type=jax.ShapeDtypeStruct((EMBED_DIM,), jnp.float32),
    scratch_types=[
        pltpu.VMEM((_NIDS,), jnp.int32),
        pltpu.VMEM((_NIDS, EMBED_DIM), jnp.float32),
        pltpu.SemaphoreType.DMA,
    ],
    compiler_params=pltpu.CompilerParams(use_tc_tiling_on_sc=False),
)
def _gather_row0(table_hbm, ids_hbm, out_hbm, idx_v, rows_v, sem):
    wid = lax.axis_index("s") * 2 + lax.axis_index("c")

    @pl.when(wid == 0)
    def _():
        pltpu.sync_copy(ids_hbm.at[pl.ds(0, _NIDS)], idx_v)
        pltpu.async_copy(table_hbm.at[idx_v], rows_v, sem).wait()
        pltpu.sync_copy(rows_v.at[0], out_hbm)


def kernel(table, ids):
    ids_flat = ids.reshape(-1).astype(jnp.int32)
    return _gather_row0(table, ids_flat)


# speedup vs baseline: 1.5171x; 1.0030x over previous
import functools

import jax
import jax.numpy as jnp
from jax import lax
from jax.experimental import pallas as pl
from jax.experimental.pallas import tpu as pltpu
from jax.experimental.pallas import tpu_sc as plsc

EMBED_DIM = 64
_NIDS = 8

_mesh = plsc.VectorSubcoreMesh(core_axis_name="c", subcore_axis_name="s")


@functools.partial(
    pl.kernel,
    mesh=_mesh,
    out_type=jax.ShapeDtypeStruct((EMBED_DIM,), jnp.float32),
    scratch_types=[
        pltpu.VMEM((_NIDS,), jnp.int32),
        pltpu.VMEM((_NIDS, EMBED_DIM), jnp.float32),
        pltpu.SemaphoreType.DMA,
    ],
    compiler_params=pltpu.CompilerParams(use_tc_tiling_on_sc=False),
)
def _gather_row0(table_hbm, ids_hbm, out_hbm, idx_v, rows_v, sem):
    wid = lax.axis_index("s") * 2 + lax.axis_index("c")

    @pl.when(wid == 0)
    def _():
        pltpu.sync_copy(ids_hbm.at[pl.ds(0, _NIDS)], idx_v)
        pltpu.async_copy(table_hbm.at[idx_v], rows_v, sem).wait()
        pltpu.sync_copy(rows_v.at[0], out_hbm)


def kernel(table, ids):
    ids_head = jax.lax.slice(ids, (0, 0), (1, _NIDS)).reshape(_NIDS)
    return _gather_row0(table, ids_head.astype(jnp.int32))


# speedup vs baseline: 2.2381x; 1.4752x over previous
import functools

import jax
import jax.numpy as jnp
from jax import lax
from jax.experimental import pallas as pl
from jax.experimental.pallas import tpu as pltpu
from jax.experimental.pallas import tpu_sc as plsc

EMBED_DIM = 64
_NIDS = 16

_mesh = plsc.VectorSubcoreMesh(core_axis_name="c", subcore_axis_name="s")


@functools.partial(
    pl.kernel,
    mesh=_mesh,
    out_type=jax.ShapeDtypeStruct((EMBED_DIM,), jnp.float32),
    scratch_types=[
        pltpu.VMEM((_NIDS,), jnp.int32),
        pltpu.VMEM((8, EMBED_DIM), jnp.float32),
    ],
)
def _gather_row0(table_hbm, ids_hbm, out_hbm, idx_v, rows_v):
    wid = lax.axis_index("s") * 2 + lax.axis_index("c")

    @pl.when(wid == 0)
    def _():
        pltpu.sync_copy(ids_hbm.at[pl.ds(0, _NIDS)], idx_v)
        idx = idx_v[...][0]
        base = pl.multiple_of((idx >> 3) << 3, 8)
        sub = idx - base
        pltpu.sync_copy(table_hbm.at[pl.ds(base, 8)], rows_v)
        pltpu.sync_copy(rows_v.at[sub], out_hbm)


def kernel(table, ids):
    ids_head = jax.lax.slice(ids, (0, 0), (1, _NIDS)).reshape(_NIDS)
    return _gather_row0(table, ids_head.astype(jnp.int32))


# speedup vs baseline: 2.3340x; 1.0429x over previous
import functools

import jax
import jax.numpy as jnp
from jax import lax
from jax.experimental import pallas as pl
from jax.experimental.pallas import tpu as pltpu
from jax.experimental.pallas import tpu_sc as plsc

EMBED_DIM = 64
_NIDS = 16

_mesh = plsc.VectorSubcoreMesh(
    core_axis_name="c", subcore_axis_name="s", num_cores=1
)


@functools.partial(
    pl.kernel,
    mesh=_mesh,
    out_type=jax.ShapeDtypeStruct((EMBED_DIM,), jnp.float32),
    scratch_types=[
        pltpu.VMEM((_NIDS,), jnp.int32),
        pltpu.VMEM((8, EMBED_DIM), jnp.float32),
    ],
)
def _gather_row0(table_hbm, ids_hbm, out_hbm, idx_v, rows_v):
    wid = lax.axis_index("s") * 2 + lax.axis_index("c")

    @pl.when(wid == 0)
    def _():
        pltpu.sync_copy(ids_hbm.at[pl.ds(0, _NIDS)], idx_v)
        idx = idx_v[...][0]
        base = pl.multiple_of((idx >> 3) << 3, 8)
        sub = idx - base
        pltpu.sync_copy(table_hbm.at[pl.ds(base, 8)], rows_v)
        pltpu.sync_copy(rows_v.at[sub], out_hbm)


def kernel(table, ids):
    ids_head = jax.lax.slice(ids, (0, 0), (1, _NIDS)).reshape(_NIDS)
    return _gather_row0(table, ids_head.astype(jnp.int32))


# speedup vs baseline: 4.3073x; 1.8455x over previous
import functools

import jax
import jax.numpy as jnp
from jax import lax
from jax.experimental import pallas as pl
from jax.experimental.pallas import tpu as pltpu
from jax.experimental.pallas import tpu_sc as plsc

EMBED_DIM = 64
_NIDS = 16
_WIN = 128

_mesh = plsc.VectorSubcoreMesh(
    core_axis_name="c", subcore_axis_name="s", num_cores=1
)


@functools.partial(
    pl.kernel,
    mesh=_mesh,
    out_type=jax.ShapeDtypeStruct((EMBED_DIM,), jnp.float32),
    scratch_types=[
        pltpu.VMEM((_NIDS,), jnp.int32),
        pltpu.VMEM((EMBED_DIM, _WIN), jnp.float32),
    ],
)
def _gather_row0(table_t_hbm, ids_hbm, out_hbm, idx_v, win_v):
    wid = lax.axis_index("s") * 2 + lax.axis_index("c")

    @pl.when(wid == 0)
    def _():
        pltpu.sync_copy(ids_hbm.at[pl.ds(0, _NIDS)], idx_v)
        idx = idx_v[...][0]
        base = pl.multiple_of((idx >> 7) << 7, _WIN)
        sub = idx - base
        pltpu.sync_copy(table_t_hbm.at[:, pl.ds(base, _WIN)], win_v)
        pltpu.sync_copy(win_v.at[:, sub], out_hbm)


def kernel(table, ids):
    ids_head = jax.lax.slice(ids, (0, 0), (1, _NIDS)).reshape(_NIDS)
    return _gather_row0(table.T, ids_head.astype(jnp.int32))


# speedup vs baseline: 6.4194x; 1.4903x over previous
import functools

import jax
import jax.numpy as jnp
from jax import lax
from jax.experimental import pallas as pl
from jax.experimental.pallas import tpu as pltpu
from jax.experimental.pallas import tpu_sc as plsc

EMBED_DIM = 64
_NIDS = 16
_WIN = 128

_mesh = plsc.VectorSubcoreMesh(
    core_axis_name="c", subcore_axis_name="s", num_cores=1
)


@functools.partial(
    pl.kernel,
    mesh=_mesh,
    out_type=jax.ShapeDtypeStruct((EMBED_DIM,), jnp.float32),
    scratch_types=[
        pltpu.VMEM((_NIDS,), jnp.int32),
        pltpu.VMEM((EMBED_DIM, _WIN), jnp.float32),
        pltpu.VMEM((EMBED_DIM,), jnp.float32),
    ],
    compiler_params=pltpu.CompilerParams(needs_layout_passes=False),
)
def _gather_row0(table_t_hbm, ids_hbm, out_hbm, idx_v, win_v, out_v):
    wid = lax.axis_index("s") * 2 + lax.axis_index("c")

    @pl.when(wid == 0)
    def _():
        pltpu.sync_copy(ids_hbm.at[pl.ds(0, _NIDS)], idx_v)
        idx = idx_v[...][0]
        base = pl.multiple_of((idx >> 7) << 7, _WIN)
        sub = idx - base
        pltpu.sync_copy(table_t_hbm.at[:, pl.ds(base, _WIN)], win_v)
        lanes = lax.broadcasted_iota(jnp.int32, (16,), 0)
        col = jnp.full((16,), 0, jnp.int32) + sub
        for r in range(EMBED_DIM // 16):
            rows = lanes + (16 * r)
            out_v[pl.ds(16 * r, 16)] = plsc.load_gather(win_v, [rows, col])
        pltpu.sync_copy(out_v, out_hbm)


def kernel(table, ids):
    ids_head = jax.lax.slice(ids, (0, 0), (1, _NIDS)).reshape(_NIDS)
    return _gather_row0(table.T, ids_head.astype(jnp.int32))


# speedup vs baseline: 6.4511x; 1.0049x over previous
import functools

import jax
import jax.numpy as jnp
from jax import lax
from jax.experimental import pallas as pl
from jax.experimental.pallas import tpu as pltpu
from jax.experimental.pallas import tpu_sc as plsc

EMBED_DIM = 64
_NIDS = 16
_WIN = 128

_mesh = plsc.VectorSubcoreMesh(
    core_axis_name="c", subcore_axis_name="s", num_cores=1
)


@functools.partial(
    pl.kernel,
    mesh=_mesh,
    out_type=jax.ShapeDtypeStruct((EMBED_DIM,), jnp.float32),
    scratch_types=[
        pltpu.VMEM((_NIDS,), jnp.int32),
        pltpu.VMEM((EMBED_DIM, _WIN), jnp.float32),
        pltpu.VMEM((EMBED_DIM,), jnp.float32),
    ],
    compiler_params=pltpu.CompilerParams(
        needs_layout_passes=False,
        skip_device_barrier=True,
        disable_bounds_checks=True,
        disable_semaphore_checks=True,
    ),
)
def _gather_row0(table_t_hbm, ids_hbm, out_hbm, idx_v, win_v, out_v):
    wid = lax.axis_index("s") * 2 + lax.axis_index("c")

    @pl.when(wid == 0)
    def _():
        pltpu.sync_copy(ids_hbm.at[pl.ds(0, _NIDS)], idx_v)
        idx = idx_v[...][0]
        base = pl.multiple_of((idx >> 7) << 7, _WIN)
        sub = idx - base
        pltpu.sync_copy(table_t_hbm.at[:, pl.ds(base, _WIN)], win_v)
        lanes = lax.broadcasted_iota(jnp.int32, (16,), 0)
        col = jnp.full((16,), 0, jnp.int32) + sub
        for r in range(EMBED_DIM // 16):
            rows = lanes + (16 * r)
            out_v[pl.ds(16 * r, 16)] = plsc.load_gather(win_v, [rows, col])
        pltpu.sync_copy(out_v, out_hbm)


def kernel(table, ids):
    ids_head = jax.lax.slice(ids, (0, 0), (1, _NIDS)).reshape(_NIDS)
    return _gather_row0(table.T, ids_head.astype(jnp.int32))


# speedup vs baseline: 6.4913x; 1.0062x over previous
import functools

import jax
import jax.numpy as jnp
from jax import lax
from jax.experimental import pallas as pl
from jax.experimental.pallas import tpu as pltpu
from jax.experimental.pallas import tpu_sc as plsc

EMBED_DIM = 64
_NIDS = 16
_WIN = 128

_mesh = plsc.VectorSubcoreMesh(
    core_axis_name="c", subcore_axis_name="s", num_cores=1
)


@functools.partial(
    pl.kernel,
    mesh=_mesh,
    out_type=jax.ShapeDtypeStruct((EMBED_DIM,), jnp.float32),
    scratch_types=[
        pltpu.VMEM((_NIDS,), jnp.int32),
        pltpu.VMEM((EMBED_DIM, _WIN), jnp.float32),
        pltpu.VMEM((EMBED_DIM,), jnp.float32),
    ],
    compiler_params=pltpu.CompilerParams(needs_layout_passes=False),
)
def _gather_row0(table_t_hbm, ids_t_hbm, out_hbm, idx_v, win_v, out_v):
    wid = lax.axis_index("s") * 2 + lax.axis_index("c")

    @pl.when(wid == 0)
    def _():
        pltpu.sync_copy(ids_t_hbm.at[0, pl.ds(0, _NIDS)], idx_v)
        idx = idx_v[...][0]
        base = pl.multiple_of((idx >> 7) << 7, _WIN)
        sub = idx - base
        pltpu.sync_copy(table_t_hbm.at[:, pl.ds(base, _WIN)], win_v)
        lanes = lax.broadcasted_iota(jnp.int32, (16,), 0)
        col = jnp.full((16,), 0, jnp.int32) + sub
        for r in range(EMBED_DIM // 16):
            rows = lanes + (16 * r)
            out_v[pl.ds(16 * r, 16)] = plsc.load_gather(win_v, [rows, col])
        pltpu.sync_copy(out_v, out_hbm)


def kernel(table, ids):
    return _gather_row0(table.T, ids.T.astype(jnp.int32))
